# baseline (speedup 1.0000x reference)
import functools

import jax
import jax.numpy as jnp
from jax import lax
from jax.experimental import pallas as pl
from jax.experimental.pallas import tpu as pltpu
from jax.experimental.pallas import tpu_sc as plsc

NC, NS = 2, 16
NW = NC * NS
BW = 128
TCH = 2
TG = 16
L = 16


@functools.partial(jax.jit, static_argnums=(2, 3, 4, 5))
def _gather(idx, table_flat, b, t, v, d):
    assert b == NW * BW and d % 8 == 0 and t % TCH == 0
    tpad = ((t + TG - 1) // TG) * TG
    full_groups = t // TG
    chunks = t // TCH
    cpg = TG // TCH
    rem_t = t - full_groups * TG
    mesh = plsc.VectorSubcoreMesh(
        core_axis_name="c", subcore_axis_name="s", num_cores=NC, num_subcores=NS
    )

    @functools.partial(
        pl.kernel,
        out_type=jax.ShapeDtypeStruct((t, d // 8, NW, 8, 128), jnp.float32),
        mesh=mesh,
        scratch_types=[
            pltpu.VMEM_SHARED((v * d,), jnp.float32),
            pltpu.VMEM((v * d,), jnp.float32),
            pltpu.VMEM((BW, tpad), jnp.int32),
            pltpu.VMEM((TG, BW), jnp.int32),
            pltpu.VMEM((TCH, d // 8, 1, 8, 128), jnp.float32),
            pltpu.VMEM((TCH, d // 8, 1, 8, 128), jnp.float32),
            pltpu.SemaphoreType.DMA,
            pltpu.SemaphoreType.DMA,
        ],
        compiler_params=pltpu.CompilerParams(
            use_tc_tiling_on_sc=False, needs_layout_passes=False,
            disable_bounds_checks=True),
    )
    def k(idx_hbm, table_hbm, out_hbm,
          table_sh, tab, idx_sl, idxT, o0, o1, so0, so1):
        sid = lax.axis_index("s")
        wid = sid * NC + lax.axis_index("c")
        base_b = wid * BW
        obufs = ((o0, so0), (o1, so1))
        ii = lax.iota(jnp.int32, L)
        zv = ii - ii

        @pl.when(sid == 0)
        def _():
            pltpu.sync_copy(table_hbm, table_sh)

        pltpu.sync_copy(
            idx_hbm.at[pl.ds(base_b, BW)], idx_sl.at[:, pl.ds(0, t)])
        plsc.subcore_barrier()
        pltpu.sync_copy(table_sh, tab)

        def tr_idx(grp16):
            def body(bb, carry):
                vals = idx_sl[bb, pl.ds(grp16 * TG, L)]
                plsc.store_scatter(idxT, [ii, zv + bb], vals)
                return carry

            lax.fori_loop(0, BW, body, 0)

        def do_t(trow, ov, cloc):
            def d8_body(d8, carry):
                dbase = d8 * 8
                for b0 in range(0, BW, L):
                    idxv = idxT[trow, pl.ds(b0, L)]
                    addrb = lax.shift_left(idxv, 6) + dbase
                    for di in range(8):
                        vals = plsc.load_gather(tab, [addrb + di])
                        ov[cloc, d8, 0, di, pl.ds(b0, L)] = vals
                return carry

            lax.fori_loop(0, d // 8, d8_body, 0)

        def do_chunk(cc, tin_grp, obi):
            ov, so = obufs[obi]

            @pl.when(cc >= 2)
            def _():
                pltpu.make_async_copy(
                    ov, out_hbm.at[pl.ds(0, TCH), :, pl.ds(0, 1)], so).wait()

            for cloc in range(TCH):
                do_t(tin_grp + cloc, ov, cloc)
            pltpu.async_copy(
                ov, out_hbm.at[pl.ds(cc * TCH, TCH), :, pl.ds(wid, 1)], so)

        def group(grp16, n_chunks):
            tr_idx(grp16)

            def sub(s, carry):
                cc = grp16 * cpg + s * 2
                do_chunk(cc, s * 2 * TCH, 0)
                do_chunk(cc + 1, (s * 2 + 1) * TCH, 1)
                return carry

            lax.fori_loop(0, n_chunks // 2, sub, 0)

        def grp_step(g, carry):
            group(g, cpg)
            return carry

        lax.fori_loop(0, full_groups, grp_step, 0)
        if rem_t:
            group(full_groups, rem_t // TCH)

        for ov, so in obufs:
            pltpu.make_async_copy(
                ov, out_hbm.at[pl.ds(0, TCH), :, pl.ds(0, 1)], so).wait()

    return k(idx, table_flat)


def kernel(segment_ids, weight):
    b, t = segment_ids.shape
    v, d = weight.shape
    out5 = _gather(
        segment_ids.astype(jnp.int32), weight.reshape(-1), b, t, v, d)
    return jnp.transpose(out5, (2, 4, 0, 1, 3)).reshape(b, t, d)


# speedup vs baseline: 1.0617x; 1.0617x over previous
import functools

import jax
import jax.numpy as jnp
from jax import lax
from jax.experimental import pallas as pl
from jax.experimental.pallas import tpu as pltpu
from jax.experimental.pallas import tpu_sc as plsc

NC, NS = 2, 16
NW = NC * NS
BW = 128
TCH = 1
TG = 16
L = 16


@functools.partial(jax.jit, static_argnums=(2, 3, 4, 5))
def _gather(idx, table_flat, b, t, v, d):
    assert b == NW * BW and d % 8 == 0 and t % TCH == 0
    tpad = ((t + TG - 1) // TG) * TG
    full_groups = t // TG
    chunks = t // TCH
    cpg = TG // TCH
    rem_t = t - full_groups * TG
    mesh = plsc.VectorSubcoreMesh(
        core_axis_name="c", subcore_axis_name="s", num_cores=NC, num_subcores=NS
    )

    @functools.partial(
        pl.kernel,
        out_type=jax.ShapeDtypeStruct((t, d // 8, NW, 8, 128), jnp.float32),
        mesh=mesh,
        scratch_types=[
            pltpu.VMEM_SHARED((v * d,), jnp.float32),
            pltpu.VMEM((v * d,), jnp.float32),
            pltpu.VMEM((BW, tpad), jnp.int32),
            pltpu.VMEM((TG, BW), jnp.int32),
            pltpu.VMEM((TCH, d // 8, 1, 8, 128), jnp.float32),
            pltpu.VMEM((TCH, d // 8, 1, 8, 128), jnp.float32),
            pltpu.SemaphoreType.DMA,
            pltpu.SemaphoreType.DMA,
        ],
        compiler_params=pltpu.CompilerParams(
            use_tc_tiling_on_sc=False, needs_layout_passes=False,
            disable_bounds_checks=True),
    )
    def k(idx_hbm, table_hbm, out_hbm,
          table_sh, tab, idx_sl, idxT, o0, o1, so0, so1):
        sid = lax.axis_index("s")
        wid = sid * NC + lax.axis_index("c")
        base_b = wid * BW
        obufs = ((o0, so0), (o1, so1))
        ii = lax.iota(jnp.int32, L)
        zv = ii - ii

        @pl.when(sid == 0)
        def _():
            pltpu.sync_copy(table_hbm, table_sh)

        pltpu.sync_copy(
            idx_hbm.at[pl.ds(base_b, BW)], idx_sl.at[:, pl.ds(0, t)])
        plsc.subcore_barrier()
        pltpu.sync_copy(table_sh, tab)

        def tr_idx(grp16):
            def body(bb, carry):
                vals = idx_sl[bb, pl.ds(grp16 * TG, L)]
                plsc.store_scatter(idxT, [ii, zv + bb], vals)
                return carry

            lax.fori_loop(0, BW, body, 0)

        def do_t(trow, ov, cloc):
            addrs = [
                lax.shift_left(idxT[trow, pl.ds(b0, L)], 6)
                for b0 in range(0, BW, L)
            ]
            for dd in range(d):
                for j, b0 in enumerate(range(0, BW, L)):
                    vals = plsc.load_gather(tab, [addrs[j] + dd])
                    ov[cloc, dd // 8, 0, dd % 8, pl.ds(b0, L)] = vals

        def do_chunk(cc, tin_grp, obi):
            ov, so = obufs[obi]

            @pl.when(cc >= 2)
            def _():
                pltpu.make_async_copy(
                    ov, out_hbm.at[pl.ds(0, TCH), :, pl.ds(0, 1)], so).wait()

            for cloc in range(TCH):
                do_t(tin_grp + cloc, ov, cloc)
            pltpu.async_copy(
                ov, out_hbm.at[pl.ds(cc * TCH, TCH), :, pl.ds(wid, 1)], so)

        def group(grp16, n_chunks):
            tr_idx(grp16)

            def sub(s, carry):
                cc = grp16 * cpg + s * 2
                do_chunk(cc, s * 2 * TCH, 0)
                do_chunk(cc + 1, (s * 2 + 1) * TCH, 1)
                return carry

            lax.fori_loop(0, n_chunks // 2, sub, 0)

        def grp_step(g, carry):
            group(g, cpg)
            return carry

        lax.fori_loop(0, full_groups, grp_step, 0)
        if rem_t:
            group(full_groups, rem_t // TCH)

        for ov, so in obufs:
            pltpu.make_async_copy(
                ov, out_hbm.at[pl.ds(0, TCH), :, pl.ds(0, 1)], so).wait()

    return k(idx, table_flat)


def kernel(segment_ids, weight):
    b, t = segment_ids.shape
    v, d = weight.shape
    out5 = _gather(
        segment_ids.astype(jnp.int32), weight.reshape(-1), b, t, v, d)
    return jnp.transpose(out5, (2, 4, 0, 1, 3)).reshape(b, t, d)


# speedup vs baseline: 1.5780x; 1.4863x over previous
import functools

import jax
import jax.numpy as jnp
from jax import lax
from jax.experimental import pallas as pl
from jax.experimental.pallas import tpu as pltpu
from jax.experimental.pallas import tpu_sc as plsc

NC, NS = 2, 16
NW = NC * NS
NB = 2


@functools.partial(jax.jit, static_argnums=(2, 3, 4, 5))
def _gather(idx, table, b, t, v, d):
    rows_per_w = b // NW
    steps = rows_per_w // NB
    assert steps % 2 == 0 and steps >= 4
    pieces = [(o, min(128, t - o)) for o in range(0, t, 128)]
    mesh = plsc.VectorSubcoreMesh(
        core_axis_name="c", subcore_axis_name="s", num_cores=NC, num_subcores=NS
    )

    @functools.partial(
        pl.kernel,
        out_type=jax.ShapeDtypeStruct((b, t, d), jnp.float32),
        mesh=mesh,
        scratch_types=[
            pltpu.VMEM_SHARED((v, d), jnp.float32),
            pltpu.VMEM((NB, t), jnp.int32),
            pltpu.VMEM((NB, t), jnp.int32),
            pltpu.VMEM((NB, t, d), jnp.float32),
            pltpu.VMEM((NB, t, d), jnp.float32),
            pltpu.SemaphoreType.DMA,
            pltpu.SemaphoreType.DMA,
            pltpu.SemaphoreType.DMA,
            pltpu.SemaphoreType.DMA,
            pltpu.SemaphoreType.DMA,
            pltpu.SemaphoreType.DMA,
        ],
        compiler_params=pltpu.CompilerParams(use_tc_tiling_on_sc=False),
    )
    def k(idx_hbm, table_hbm, out_hbm,
          table_sh, idx0, idx1, rows0, rows1, si0, si1, sg0, sg1, so0, so1):
        sid = lax.axis_index("s")
        wid = sid * NC + lax.axis_index("c")
        base_row = wid * rows_per_w
        bufs = ((idx0, rows0, si0, sg0, so0), (idx1, rows1, si1, sg1, so1))

        @pl.when(sid == 0)
        def _():
            pltpu.sync_copy(table_hbm, table_sh)

        plsc.subcore_barrier()

        def issue_idx(g, bf):
            idx_v, _, si, _, _ = bufs[bf]
            pltpu.async_copy(idx_hbm.at[pl.ds(base_row + g * NB, NB)], idx_v, si)

        def run_chunk(g, bf, wait_out, next_idx):
            idx_v, rows_v, si, sg, so = bufs[bf]
            pltpu.make_async_copy(idx_hbm.at[pl.ds(0, NB)], idx_v, si).wait()
            if wait_out:
                pltpu.make_async_copy(
                    rows_v, out_hbm.at[pl.ds(0, NB)], so).wait()
            copies = [
                pltpu.async_copy(
                    table_sh.at[idx_v.at[r, pl.ds(o, n)]],
                    rows_v.at[r, pl.ds(o, n)],
                    sg,
                )
                for r in range(NB)
                for (o, n) in pieces
            ]
            for c in copies:
                c.wait()
            if next_idx:
                issue_idx(g + 2, bf)
            pltpu.async_copy(
                rows_v, out_hbm.at[pl.ds(base_row + g * NB, NB)], so)

        issue_idx(0, 0)
        issue_idx(1, 1)
        run_chunk(0, 0, wait_out=False, next_idx=True)
        run_chunk(1, 1, wait_out=False, next_idx=True)

        def outer(o, carry):
            g = o * 2
            run_chunk(g, 0, wait_out=True, next_idx=True)
            run_chunk(g + 1, 1, wait_out=True, next_idx=True)
            return carry

        lax.fori_loop(1, steps // 2 - 1, outer, 0)

        run_chunk(steps - 2, 0, wait_out=True, next_idx=False)
        run_chunk(steps - 1, 1, wait_out=True, next_idx=False)
        for bf in (0, 1):
            _, rows_v, _, _, so = bufs[bf]
            pltpu.make_async_copy(
                rows_v, out_hbm.at[pl.ds(0, NB)], so).wait()

    return k(idx, table)


def kernel(segment_ids, weight):
    b, t = segment_ids.shape
    v, d = weight.shape
    out = _gather(segment_ids.astype(jnp.int32), weight, b, t, v, d)
    one = 1.0 + 0.0 * weight[0, 0]
    return out * one
